# baseline (speedup 1.0000x reference)
import functools

import jax
import jax.numpy as jnp
from jax import lax
from jax.experimental import pallas as pl
from jax.experimental.pallas import tpu as pltpu
from jax.experimental.pallas import tpu_sc as plsc

_NC = 2
_NS = 16
_NW = _NC * _NS


def _sc_gather_fn(n_tokens, hidden):
    n_per_w = n_tokens // _NW
    sizes = []
    rem = n_per_w
    while rem:
        sz = min(128, rem)
        sizes.append(sz)
        rem -= sz
    offs = [sum(sizes[:i]) for i in range(len(sizes))]

    mesh = plsc.VectorSubcoreMesh(core_axis_name="c", subcore_axis_name="s")

    @functools.partial(
        pl.kernel,
        out_type=jax.ShapeDtypeStruct((n_tokens, hidden), jnp.float32),
        mesh=mesh,
        scratch_types=[
            pltpu.VMEM((n_per_w,), jnp.int32),
            pltpu.VMEM((128, hidden), jnp.float32),
            pltpu.VMEM((128, hidden), jnp.float32),
            pltpu.SemaphoreType.DMA,
            pltpu.SemaphoreType.DMA,
        ],
    )
    def sc_gather(idx_hbm, table_hbm, x_hbm, idx_v, rows0, rows1, sem0, sem1):
        wid = lax.axis_index("s") * _NC + lax.axis_index("c")
        base = wid * n_per_w
        pltpu.sync_copy(idx_hbm.at[pl.ds(base, n_per_w)], idx_v)
        rows = (rows0, rows1)
        sems = (sem0, sem1)
        n = len(sizes)

        def start(c):
            sz = sizes[c]
            return pltpu.async_copy(
                table_hbm.at[idx_v.at[pl.ds(offs[c], sz)]],
                rows[c % 2].at[pl.ds(0, sz)],
                sems[c % 2],
            )

        pending = start(0)
        for c in range(n):
            nxt = start(c + 1) if c + 1 < n else None
            pending.wait()
            pltpu.sync_copy(
                rows[c % 2].at[pl.ds(0, sizes[c])],
                x_hbm.at[pl.ds(base + offs[c], sizes[c])],
            )
            pending = nxt

    return sc_gather


def _tc_matmul_body(x_ref, w_ref, b_ref, o_ref):
    acc = lax.dot_general(
        w_ref[...].astype(jnp.bfloat16),
        x_ref[0].astype(jnp.bfloat16),
        (((0,), (1,)), ((), ())),
        preferred_element_type=jnp.float32,
    )
    o_ref[0] = acc + b_ref[...]


def _tc_matmul_body_aliased(x_ref, w_ref, b_ref, prev_ref, o_ref):
    del prev_ref
    _tc_matmul_body(x_ref, w_ref, b_ref, o_ref)


def _tc_matmul_t_slice(x_t, w, b_col, prev, l_off, seqlen_total):
    n_l, bsz, hidden = x_t.shape
    vocab = w.shape[1]
    specs = [
        pl.BlockSpec((1, bsz, hidden), lambda i: (i, 0, 0)),
        pl.BlockSpec((hidden, vocab), lambda i: (0, 0)),
        pl.BlockSpec((vocab, 1), lambda i: (0, 0)),
    ]
    args = [x_t, w, b_col]
    body = _tc_matmul_body
    aliases = {}
    if prev is not None:
        specs.append(pl.BlockSpec(memory_space=pl.ANY))
        args.append(prev)
        body = _tc_matmul_body_aliased
        aliases = {3: 0}
    return pl.pallas_call(
        body,
        grid=(n_l,),
        in_specs=specs,
        out_specs=pl.BlockSpec(
            (1, vocab, bsz), lambda i, l_off=l_off: (i + l_off, 0, 0)
        ),
        out_shape=jax.ShapeDtypeStruct((seqlen_total, vocab, bsz), jnp.float32),
        input_output_aliases=aliases,
    )(*args)


@jax.jit
def kernel(input_ids, embedding, W, b):
    bsz, seqlen = input_ids.shape
    vocab, hidden = embedding.shape
    n_tokens = bsz * seqlen

    ids_t = input_ids.T.reshape(-1).astype(jnp.int32)
    l_parts = [5, 15, 30]
    assert sum(l_parts) == seqlen
    b_col = b.reshape(-1, 1)
    xs = []
    l_off = 0
    for l_part in l_parts:
        tok_part = l_part * bsz
        xs.append(
            _sc_gather_fn(tok_part, hidden)(
                lax.dynamic_slice_in_dim(ids_t, l_off * bsz, tok_part), embedding
            )
        )
        l_off += l_part
    out_t = None
    l_off = 0
    for l_part, x_s in zip(l_parts, xs):
        out_t = _tc_matmul_t_slice(
            x_s.reshape(l_part, bsz, hidden), W, b_col, out_t, l_off, seqlen
        )
        l_off += l_part
    return jnp.transpose(out_t, (2, 0, 1))


# speedup vs baseline: 1.0225x; 1.0225x over previous
import functools

import jax
import jax.numpy as jnp
from jax import lax
from jax.experimental import pallas as pl
from jax.experimental.pallas import tpu as pltpu
from jax.experimental.pallas import tpu_sc as plsc

_NC = 2
_NS = 16
_NW = _NC * _NS


def _sc_gather_fn(n_tokens, hidden):
    n_per_w = n_tokens // _NW
    sizes = []
    rem = n_per_w
    while rem:
        sz = min(128, rem)
        sizes.append(sz)
        rem -= sz
    offs = [sum(sizes[:i]) for i in range(len(sizes))]

    mesh = plsc.VectorSubcoreMesh(core_axis_name="c", subcore_axis_name="s")

    nbuf = 4

    @functools.partial(
        pl.kernel,
        out_type=jax.ShapeDtypeStruct((n_tokens, hidden), jnp.float32),
        mesh=mesh,
        scratch_types=[
            pltpu.VMEM((n_per_w,), jnp.int32),
        ]
        + [pltpu.VMEM((128, hidden), jnp.float32)] * nbuf
        + [pltpu.SemaphoreType.DMA] * (2 * nbuf),
    )
    def sc_gather(idx_hbm, table_hbm, x_hbm, idx_v, *bufs_sems):
        rows = bufs_sems[:nbuf]
        gsem = bufs_sems[nbuf : 2 * nbuf]
        ssem = bufs_sems[2 * nbuf :]
        wid = lax.axis_index("s") * _NC + lax.axis_index("c")
        base = wid * n_per_w
        pltpu.sync_copy(idx_hbm.at[pl.ds(base, n_per_w)], idx_v)
        n = len(sizes)

        def start_gather(c):
            sz = sizes[c]
            return pltpu.async_copy(
                table_hbm.at[idx_v.at[pl.ds(offs[c], sz)]],
                rows[c % nbuf].at[pl.ds(0, sz)],
                gsem[c % nbuf],
            )

        def start_store(c):
            sz = sizes[c]
            return pltpu.async_copy(
                rows[c % nbuf].at[pl.ds(0, sz)],
                x_hbm.at[pl.ds(base + offs[c], sz)],
                ssem[c % nbuf],
            )

        gathers = {}
        stores = {}
        for j in range(min(2, n)):
            gathers[j] = start_gather(j)
        for c in range(n):
            nxt = c + 2
            if nxt < n:
                if nxt - nbuf >= 0:
                    stores[nxt - nbuf].wait()
                gathers[nxt] = start_gather(nxt)
            gathers[c].wait()
            stores[c] = start_store(c)
        for c in range(max(0, n - nbuf), n):
            if c in stores:
                stores[c].wait()

    return sc_gather


def _tc_matmul_body(x_ref, w_ref, b_ref, o_ref):
    acc = lax.dot_general(
        w_ref[...].astype(jnp.bfloat16),
        x_ref[0].astype(jnp.bfloat16),
        (((0,), (1,)), ((), ())),
        preferred_element_type=jnp.float32,
    )
    o_ref[0] = acc + b_ref[...]


def _tc_matmul_body_aliased(x_ref, w_ref, b_ref, prev_ref, o_ref):
    del prev_ref
    _tc_matmul_body(x_ref, w_ref, b_ref, o_ref)


def _tc_matmul_t_slice(x_t, w, b_col, prev, l_off, seqlen_total):
    n_l, bsz, hidden = x_t.shape
    vocab = w.shape[1]
    specs = [
        pl.BlockSpec((1, bsz, hidden), lambda i: (i, 0, 0)),
        pl.BlockSpec((hidden, vocab), lambda i: (0, 0)),
        pl.BlockSpec((vocab, 1), lambda i: (0, 0)),
    ]
    args = [x_t, w, b_col]
    body = _tc_matmul_body
    aliases = {}
    if prev is not None:
        specs.append(pl.BlockSpec(memory_space=pl.ANY))
        args.append(prev)
        body = _tc_matmul_body_aliased
        aliases = {3: 0}
    return pl.pallas_call(
        body,
        grid=(n_l,),
        in_specs=specs,
        out_specs=pl.BlockSpec(
            (1, vocab, bsz), lambda i, l_off=l_off: (i + l_off, 0, 0)
        ),
        out_shape=jax.ShapeDtypeStruct((seqlen_total, vocab, bsz), jnp.float32),
        input_output_aliases=aliases,
    )(*args)


@jax.jit
def kernel(input_ids, embedding, W, b):
    bsz, seqlen = input_ids.shape
    vocab, hidden = embedding.shape
    n_tokens = bsz * seqlen

    ids_t = input_ids.T.reshape(-1).astype(jnp.int32)
    l_parts = [15, 35]
    assert sum(l_parts) == seqlen
    b_col = b.reshape(-1, 1)
    xs = []
    l_off = 0
    for l_part in l_parts:
        tok_part = l_part * bsz
        xs.append(
            _sc_gather_fn(tok_part, hidden)(
                lax.dynamic_slice_in_dim(ids_t, l_off * bsz, tok_part), embedding
            )
        )
        l_off += l_part
    out_t = None
    l_off = 0
    for l_part, x_s in zip(l_parts, xs):
        out_t = _tc_matmul_t_slice(
            x_s.reshape(l_part, bsz, hidden), W, b_col, out_t, l_off, seqlen
        )
        l_off += l_part
    return jnp.transpose(out_t, (2, 0, 1))


# speedup vs baseline: 1.1799x; 1.1539x over previous
import functools

import jax
import jax.numpy as jnp
from jax import lax
from jax.experimental import pallas as pl
from jax.experimental.pallas import tpu as pltpu
from jax.experimental.pallas import tpu_sc as plsc

_NC = 2
_NS = 16
_NW = _NC * _NS


def _sc_gather_fn(n_tokens, vocab, hidden):
    n_per_w = n_tokens // _NW
    sizes = []
    rem = n_per_w
    while rem:
        sz = min(128, rem)
        sizes.append(sz)
        rem -= sz
    offs = [sum(sizes[:i]) for i in range(len(sizes))]

    mesh = plsc.VectorSubcoreMesh(core_axis_name="c", subcore_axis_name="s")

    nbuf = 4

    @functools.partial(
        pl.kernel,
        out_type=jax.ShapeDtypeStruct((n_tokens, hidden), jnp.float32),
        mesh=mesh,
        scratch_types=[
            pltpu.VMEM((n_per_w,), jnp.int32),
            pltpu.VMEM_SHARED((vocab, hidden), jnp.float32),
        ]
        + [pltpu.VMEM((128, hidden), jnp.float32)] * nbuf
        + [pltpu.SemaphoreType.DMA] * (2 * nbuf),
    )
    def sc_gather(idx_hbm, table_hbm, x_hbm, idx_v, table_sp, *bufs_sems):
        rows = bufs_sems[:nbuf]
        gsem = bufs_sems[nbuf : 2 * nbuf]
        ssem = bufs_sems[2 * nbuf :]
        sid = lax.axis_index("s")
        wid = sid * _NC + lax.axis_index("c")
        base = wid * n_per_w

        @pl.when(sid == 0)
        def _():
            pltpu.sync_copy(table_hbm, table_sp)

        pltpu.sync_copy(idx_hbm.at[pl.ds(base, n_per_w)], idx_v)
        plsc.subcore_barrier()
        n = len(sizes)

        def start_gather(c):
            sz = sizes[c]
            return pltpu.async_copy(
                table_sp.at[idx_v.at[pl.ds(offs[c], sz)]],
                rows[c % nbuf].at[pl.ds(0, sz)],
                gsem[c % nbuf],
            )

        def start_store(c):
            sz = sizes[c]
            return pltpu.async_copy(
                rows[c % nbuf].at[pl.ds(0, sz)],
                x_hbm.at[pl.ds(base + offs[c], sz)],
                ssem[c % nbuf],
            )

        gathers = {}
        stores = {}
        for j in range(min(2, n)):
            gathers[j] = start_gather(j)
        for c in range(n):
            nxt = c + 2
            if nxt < n:
                if nxt - nbuf >= 0:
                    stores[nxt - nbuf].wait()
                gathers[nxt] = start_gather(nxt)
            gathers[c].wait()
            stores[c] = start_store(c)
        for c in range(max(0, n - nbuf), n):
            if c in stores:
                stores[c].wait()

    return sc_gather


def _tc_matmul_body(x_ref, wt_ref, b_ref, o_ref):
    acc = lax.dot_general(
        wt_ref[...],
        x_ref[0],
        (((1,), (1,)), ((), ())),
        preferred_element_type=jnp.float32,
    )
    o_ref[0] = acc + b_ref[...]


def _tc_matmul_body_aliased(x_ref, w_ref, b_ref, prev_ref, o_ref):
    del prev_ref
    _tc_matmul_body(x_ref, w_ref, b_ref, o_ref)


def _tc_matmul_t_slice(x_t, wt, b_col, prev, l_off, seqlen_total):
    n_l, bsz, hidden = x_t.shape
    vocab = wt.shape[0]
    specs = [
        pl.BlockSpec((1, bsz, hidden), lambda i: (i, 0, 0)),
        pl.BlockSpec((vocab, hidden), lambda i: (0, 0)),
        pl.BlockSpec((vocab, 1), lambda i: (0, 0)),
    ]
    args = [x_t, wt, b_col]
    body = _tc_matmul_body
    aliases = {}
    if prev is not None:
        specs.append(pl.BlockSpec(memory_space=pl.ANY))
        args.append(prev)
        body = _tc_matmul_body_aliased
        aliases = {3: 0}
    return pl.pallas_call(
        body,
        grid=(n_l,),
        in_specs=specs,
        out_specs=pl.BlockSpec(
            (1, vocab, bsz), lambda i, l_off=l_off: (i + l_off, 0, 0)
        ),
        out_shape=jax.ShapeDtypeStruct((seqlen_total, vocab, bsz), jnp.float32),
        input_output_aliases=aliases,
    )(*args)


@jax.jit
def kernel(input_ids, embedding, W, b):
    bsz, seqlen = input_ids.shape
    vocab, hidden = embedding.shape
    n_tokens = bsz * seqlen

    ids_t = input_ids.T.reshape(-1).astype(jnp.int32)
    l_parts = [15, 35]
    assert sum(l_parts) == seqlen
    b_col = b.reshape(-1, 1)
    xs = []
    l_off = 0
    for l_part in l_parts:
        tok_part = l_part * bsz
        xs.append(
            _sc_gather_fn(tok_part, vocab, hidden)(
                lax.dynamic_slice_in_dim(ids_t, l_off * bsz, tok_part), embedding
            )
        )
        l_off += l_part
    Wt = W.T
    out_t = None
    l_off = 0
    for l_part, x_s in zip(l_parts, xs):
        out_t = _tc_matmul_t_slice(
            x_s.reshape(l_part, bsz, hidden), Wt, b_col, out_t, l_off, seqlen
        )
        l_off += l_part
    return jnp.transpose(out_t, (2, 0, 1))


# speedup vs baseline: 1.1864x; 1.0055x over previous
import functools

import jax
import jax.numpy as jnp
from jax import lax
from jax.experimental import pallas as pl
from jax.experimental.pallas import tpu as pltpu
from jax.experimental.pallas import tpu_sc as plsc

_NC = 2
_NS = 16
_NW = _NC * _NS


def _sc_gather_fn(n_tokens, vocab, hidden):
    n_per_w = n_tokens // _NW
    sizes = []
    rem = n_per_w
    while rem:
        sz = min(128, rem)
        sizes.append(sz)
        rem -= sz
    offs = [sum(sizes[:i]) for i in range(len(sizes))]

    mesh = plsc.VectorSubcoreMesh(core_axis_name="c", subcore_axis_name="s")

    nbuf = 4

    @functools.partial(
        pl.kernel,
        out_type=jax.ShapeDtypeStruct((n_tokens, hidden), jnp.float32),
        mesh=mesh,
        scratch_types=[
            pltpu.VMEM((n_per_w,), jnp.int32),
            pltpu.VMEM_SHARED((vocab, hidden), jnp.float32),
        ]
        + [pltpu.VMEM((128, hidden), jnp.float32)] * nbuf
        + [pltpu.SemaphoreType.DMA] * (2 * nbuf),
    )
    def sc_gather(idx_hbm, table_hbm, x_hbm, idx_v, table_sp, *bufs_sems):
        rows = bufs_sems[:nbuf]
        gsem = bufs_sems[nbuf : 2 * nbuf]
        ssem = bufs_sems[2 * nbuf :]
        sid = lax.axis_index("s")
        wid = sid * _NC + lax.axis_index("c")
        base = wid * n_per_w

        @pl.when(sid == 0)
        def _():
            pltpu.sync_copy(table_hbm, table_sp)

        pltpu.sync_copy(idx_hbm.at[pl.ds(base, n_per_w)], idx_v)
        plsc.subcore_barrier()
        n = len(sizes)

        def start_gather(c):
            sz = sizes[c]
            return pltpu.async_copy(
                table_sp.at[idx_v.at[pl.ds(offs[c], sz)]],
                rows[c % nbuf].at[pl.ds(0, sz)],
                gsem[c % nbuf],
            )

        def start_store(c):
            sz = sizes[c]
            return pltpu.async_copy(
                rows[c % nbuf].at[pl.ds(0, sz)],
                x_hbm.at[pl.ds(base + offs[c], sz)],
                ssem[c % nbuf],
            )

        gathers = {}
        stores = {}
        for j in range(min(2, n)):
            gathers[j] = start_gather(j)
        for c in range(n):
            nxt = c + 2
            if nxt < n:
                if nxt - nbuf >= 0:
                    stores[nxt - nbuf].wait()
                gathers[nxt] = start_gather(nxt)
            gathers[c].wait()
            stores[c] = start_store(c)
        for c in range(max(0, n - nbuf), n):
            if c in stores:
                stores[c].wait()

    return sc_gather


def _tc_matmul_body(x_ref, wt_ref, b_ref, o_ref):
    acc = lax.dot_general(
        wt_ref[...],
        x_ref[0],
        (((1,), (1,)), ((), ())),
        preferred_element_type=jnp.float32,
    )
    o_ref[0] = acc + b_ref[...]


def _tc_matmul_body_aliased(x_ref, w_ref, b_ref, prev_ref, o_ref):
    del prev_ref
    _tc_matmul_body(x_ref, w_ref, b_ref, o_ref)


def _tc_matmul_t_slice(x_t, wt, b_col, prev, l_off, seqlen_total):
    n_l, bsz, hidden = x_t.shape
    vocab = wt.shape[0]
    specs = [
        pl.BlockSpec((1, bsz, hidden), lambda i: (i, 0, 0)),
        pl.BlockSpec((vocab, hidden), lambda i: (0, 0)),
        pl.BlockSpec((vocab, 1), lambda i: (0, 0)),
    ]
    args = [x_t, wt, b_col]
    body = _tc_matmul_body
    aliases = {}
    if prev is not None:
        specs.append(pl.BlockSpec(memory_space=pl.ANY))
        args.append(prev)
        body = _tc_matmul_body_aliased
        aliases = {3: 0}
    return pl.pallas_call(
        body,
        grid=(n_l,),
        in_specs=specs,
        out_specs=pl.BlockSpec(
            (1, vocab, bsz), lambda i, l_off=l_off: (i + l_off, 0, 0)
        ),
        out_shape=jax.ShapeDtypeStruct((seqlen_total, vocab, bsz), jnp.float32),
        input_output_aliases=aliases,
    )(*args)


@jax.jit
def kernel(input_ids, embedding, W, b):
    bsz, seqlen = input_ids.shape
    vocab, hidden = embedding.shape
    n_tokens = bsz * seqlen

    ids_t = input_ids.T.reshape(-1).astype(jnp.int32)
    l_parts = [10, 40]
    assert sum(l_parts) == seqlen
    b_col = b.reshape(-1, 1)
    xs = []
    l_off = 0
    for l_part in l_parts:
        tok_part = l_part * bsz
        xs.append(
            _sc_gather_fn(tok_part, vocab, hidden)(
                lax.dynamic_slice_in_dim(ids_t, l_off * bsz, tok_part), embedding
            )
        )
        l_off += l_part
    Wt = W.T
    out_t = None
    l_off = 0
    for l_part, x_s in zip(l_parts, xs):
        out_t = _tc_matmul_t_slice(
            x_s.reshape(l_part, bsz, hidden), Wt, b_col, out_t, l_off, seqlen
        )
        l_off += l_part
    return jnp.transpose(out_t, (2, 0, 1))
